# baseline (speedup 1.0000x reference)
import functools

import jax
import jax.numpy as jnp
from jax import lax
from jax.experimental import pallas as pl
from jax.experimental.pallas import tpu as pltpu

_VMEM_LIMIT = 48 * 1024 * 1024
_DT = jnp.bfloat16


def _row_tile(h, target):
    t = max(1, min(target, h))
    while h % t:
        t -= 1
    return t


def _c11_body(x_ref, w_ref, b_ref, o_ref):
    y = jnp.dot(x_ref[0, 0], w_ref[...], preferred_element_type=jnp.float32)
    o_ref[0, 0] = jnp.maximum(y + b_ref[...], 0.0).astype(o_ref.dtype)


def _conv1_1(x, w, b):
    n, h, wd, cin = x.shape
    cout = w.shape[-1]
    xp = jnp.pad(x, ((0, 0), (1, 1), (1, 1), (0, 0)))
    x9 = jnp.concatenate(
        [xp[:, dy:dy + h, dx:dx + wd, :] for dy in range(3) for dx in range(3)],
        axis=-1)
    th = _row_tile(h, 64)
    nr = h // th
    m = th * wd
    x_flat = x9.reshape(n, nr, m, 9 * cin)
    wf = w.reshape(9 * cin, cout)
    out = pl.pallas_call(
        _c11_body,
        out_shape=jax.ShapeDtypeStruct((n, nr, m, cout), _DT),
        grid=(n, nr),
        in_specs=[
            pl.BlockSpec((1, 1, m, 9 * cin), lambda bi, r: (bi, r, 0, 0)),
            pl.BlockSpec((9 * cin, cout), lambda bi, r: (0, 0)),
            pl.BlockSpec((1, cout), lambda bi, r: (0, 0)),
        ],
        out_specs=pl.BlockSpec((1, 1, m, cout), lambda bi, r: (bi, r, 0, 0)),
        compiler_params=pltpu.CompilerParams(
            dimension_semantics=("parallel", "parallel"),
            vmem_limit_bytes=_VMEM_LIMIT),
    )(x_flat, wf, b.reshape(1, cout).astype(jnp.float32))
    return out.reshape(n, h, wd, cout)


def _conv9_body(x_ref, w_ref, b_ref, o_ref, *, m, wp8, cout, th, wd, pool,
                gram_scale):
    x = x_ref[0, 0]
    xc = jnp.concatenate(
        [x[dy * wp8 + dx:dy * wp8 + dx + m, :]
         for dy in range(3) for dx in range(3)], axis=1)
    y = jnp.dot(xc, w_ref[...], preferred_element_type=jnp.float32)
    y = jnp.maximum(y + b_ref[...], 0.0)
    if gram_scale is not None:
        col = lax.broadcasted_iota(jnp.int32, (m, cout), 0) % wp8
        ym = jnp.where(col < wd, y, 0.0).astype(_DT)
        g = lax.dot_general(ym, ym, (((0,), (0,)), ((), ())),
                            preferred_element_type=jnp.float32)
        o_ref[0] = g * gram_scale
        return
    if pool:
        y2 = y.reshape(th // 2, 2 * wp8, cout)
        y = jnp.maximum(y2[:, :wp8, :], y2[:, wp8:, :])
        y = y.reshape((th // 2) * wp8, cout)
    o_ref[0, 0] = y.astype(o_ref.dtype)


def _conv_general(x, w, b, *, pool=False, gram=False, th_target=64):
    n, h, wd, cin = x.shape
    cout = w.shape[-1]
    th = _row_tile(h, th_target)
    nr = h // th
    wp8 = wd + 8
    rwin = th + 3
    m = th * wp8
    l = rwin * wp8

    xp = jnp.pad(x, ((0, 0), (1, 2), (1, 7), (0, 0)))
    xwin = jnp.stack([xp[:, r * th:r * th + rwin] for r in range(nr)], axis=1)
    x_flat = xwin.reshape(n, nr, l, cin)
    bias = b.reshape(1, cout).astype(jnp.float32)
    wf = w.reshape(9 * cin, cout)

    if gram:
        assert nr == 1
        scale = 1.0 / float(cout * h * wd)
        body = functools.partial(_conv9_body, m=m, wp8=wp8, cout=cout, th=th,
                                 wd=wd, pool=False, gram_scale=scale)
        return pl.pallas_call(
            body,
            out_shape=jax.ShapeDtypeStruct((n, cout, cout), jnp.float32),
            grid=(n,),
            in_specs=[
                pl.BlockSpec((1, 1, l, cin), lambda bi: (bi, 0, 0, 0)),
                pl.BlockSpec((9 * cin, cout), lambda bi: (0, 0)),
                pl.BlockSpec((1, cout), lambda bi: (0, 0)),
            ],
            out_specs=pl.BlockSpec((1, cout, cout), lambda bi: (bi, 0, 0)),
            compiler_params=pltpu.CompilerParams(
                dimension_semantics=("parallel",),
                vmem_limit_bytes=_VMEM_LIMIT),
        )(x_flat, wf, bias)

    if pool:
        mo = (th // 2) * wp8
    else:
        mo = m

    body = functools.partial(_conv9_body, m=m, wp8=wp8, cout=cout, th=th,
                             wd=wd, pool=pool, gram_scale=None)
    out = pl.pallas_call(
        body,
        out_shape=jax.ShapeDtypeStruct((n, nr, mo, cout), _DT),
        grid=(n, nr),
        in_specs=[
            pl.BlockSpec((1, 1, l, cin), lambda bi, r: (bi, r, 0, 0)),
            pl.BlockSpec((9 * cin, cout), lambda bi, r: (0, 0)),
            pl.BlockSpec((1, cout), lambda bi, r: (0, 0)),
        ],
        out_specs=pl.BlockSpec((1, 1, mo, cout), lambda bi, r: (bi, r, 0, 0)),
        compiler_params=pltpu.CompilerParams(
            dimension_semantics=("parallel", "parallel"),
            vmem_limit_bytes=_VMEM_LIMIT),
    )(x_flat, wf, bias)

    if pool:
        yr = out.reshape(n, h // 2, wp8, cout)
        return jnp.maximum(yr[:, :, 0:wd:2, :], yr[:, :, 1:wd:2, :])
    return out.reshape(n, h, wp8, cout)[:, :, :wd, :]


def kernel(x_nchw, w0, b0, w1, b1, w2, b2, w3, b3, w4, b4, w5, b5, w6, b6,
           w7, b7, w8, b8):
    x = jnp.transpose(x_nchw, (0, 2, 3, 1)).astype(_DT)
    cast = lambda w: w.astype(_DT)

    x = _conv1_1(x, cast(w0), b0)
    x = _conv_general(x, cast(w1), b1, pool=True)
    x = _conv_general(x, cast(w2), b2)
    x = _conv_general(x, cast(w3), b3, pool=True)
    x = _conv_general(x, cast(w4), b4)
    x = _conv_general(x, cast(w5), b5)
    x = _conv_general(x, cast(w6), b6)
    x = _conv_general(x, cast(w7), b7, pool=True)
    g = _conv_general(x, cast(w8), b8, gram=True)
    return [g]


# speedup vs baseline: 1.3151x; 1.3151x over previous
import functools

import jax
import jax.numpy as jnp
from jax import lax
from jax.experimental import pallas as pl
from jax.experimental.pallas import tpu as pltpu

_VMEM_LIMIT = 48 * 1024 * 1024
_DT = jnp.bfloat16


def _row_tile(h, target):
    t = max(1, min(target, h))
    while h % t:
        t -= 1
    return t


def _c11_body(x_ref, w_ref, b_ref, o_ref):
    y = jnp.dot(x_ref[0, 0], w_ref[...], preferred_element_type=jnp.float32)
    o_ref[0, 0] = jnp.maximum(y + b_ref[...], 0.0).astype(o_ref.dtype)


def _conv1_1(x, w, b):
    n, h, wd, cin = x.shape
    cout = w.shape[-1]
    xp = jnp.pad(x, ((0, 0), (1, 1), (1, 1), (0, 0)))
    x9 = jnp.concatenate(
        [xp[:, dy:dy + h, dx:dx + wd, :] for dy in range(3) for dx in range(3)],
        axis=-1)
    th = _row_tile(h, 64)
    nr = h // th
    m = th * wd
    x_flat = x9.reshape(n, nr, m, 9 * cin)
    wf = w.reshape(9 * cin, cout)
    out = pl.pallas_call(
        _c11_body,
        out_shape=jax.ShapeDtypeStruct((n, nr, m, cout), _DT),
        grid=(n, nr),
        in_specs=[
            pl.BlockSpec((1, 1, m, 9 * cin), lambda bi, r: (bi, r, 0, 0)),
            pl.BlockSpec((9 * cin, cout), lambda bi, r: (0, 0)),
            pl.BlockSpec((1, cout), lambda bi, r: (0, 0)),
        ],
        out_specs=pl.BlockSpec((1, 1, m, cout), lambda bi, r: (bi, r, 0, 0)),
        compiler_params=pltpu.CompilerParams(
            dimension_semantics=("parallel", "parallel"),
            vmem_limit_bytes=_VMEM_LIMIT),
    )(x_flat, wf, b.reshape(1, cout).astype(jnp.float32))
    return out.reshape(n, h, wd, cout)


def _pool_body(x_ref, o_ref, *, c):
    v = jnp.maximum(x_ref[0, :, 0], x_ref[0, :, 1])
    o_ref[0] = jnp.maximum(v[..., :c], v[..., c:])


def _maxpool2(xq):
    n, h2, _, wp4, c2 = xq.shape
    c = c2 // 2
    th2 = _row_tile(h2, 128)
    nr2 = h2 // th2
    return pl.pallas_call(
        functools.partial(_pool_body, c=c),
        out_shape=jax.ShapeDtypeStruct((n, h2, wp4, c), xq.dtype),
        grid=(n, nr2),
        in_specs=[pl.BlockSpec((1, th2, 2, wp4, c2),
                               lambda bi, r: (bi, r, 0, 0, 0))],
        out_specs=pl.BlockSpec((1, th2, wp4, c), lambda bi, r: (bi, r, 0, 0)),
        compiler_params=pltpu.CompilerParams(
            dimension_semantics=("parallel", "parallel"),
            vmem_limit_bytes=_VMEM_LIMIT),
    )(xq)


def _conv9_body(x_ref, w_ref, b_ref, o_ref, *, m, wp8, cout, th, wd,
                gram_scale):
    x = x_ref[0, 0]
    xc = jnp.concatenate(
        [x[dy * wp8 + dx:dy * wp8 + dx + m, :]
         for dy in range(3) for dx in range(3)], axis=1)
    y = jnp.dot(xc, w_ref[...], preferred_element_type=jnp.float32)
    y = jnp.maximum(y + b_ref[...], 0.0)
    if gram_scale is not None:
        col = lax.broadcasted_iota(jnp.int32, (m, cout), 0) % wp8
        ym = jnp.where(col < wd, y, 0.0).astype(_DT)
        g = lax.dot_general(ym, ym, (((0,), (0,)), ((), ())),
                            preferred_element_type=jnp.float32)
        o_ref[0] = g * gram_scale
        return
    o_ref[0, 0] = y.astype(o_ref.dtype)


def _conv_general(x, w, b, *, pool=False, gram=False, th_target=64):
    n, h, wd, cin = x.shape
    cout = w.shape[-1]
    th = _row_tile(h, th_target)
    nr = h // th
    wp8 = wd + 8
    rwin = th + 3
    m = th * wp8
    l = rwin * wp8

    xp = jnp.pad(x, ((0, 0), (1, 2), (1, 7), (0, 0)))
    xwin = jnp.stack([xp[:, r * th:r * th + rwin] for r in range(nr)], axis=1)
    x_flat = xwin.reshape(n, nr, l, cin)
    bias = b.reshape(1, cout).astype(jnp.float32)
    wf = w.reshape(9 * cin, cout)

    if gram:
        assert nr == 1
        scale = 1.0 / float(cout * h * wd)
        body = functools.partial(_conv9_body, m=m, wp8=wp8, cout=cout, th=th,
                                 wd=wd, gram_scale=scale)
        return pl.pallas_call(
            body,
            out_shape=jax.ShapeDtypeStruct((n, cout, cout), jnp.float32),
            grid=(n,),
            in_specs=[
                pl.BlockSpec((1, 1, l, cin), lambda bi: (bi, 0, 0, 0)),
                pl.BlockSpec((9 * cin, cout), lambda bi: (0, 0)),
                pl.BlockSpec((1, cout), lambda bi: (0, 0)),
            ],
            out_specs=pl.BlockSpec((1, cout, cout), lambda bi: (bi, 0, 0)),
            compiler_params=pltpu.CompilerParams(
                dimension_semantics=("parallel",),
                vmem_limit_bytes=_VMEM_LIMIT),
        )(x_flat, wf, bias)

    body = functools.partial(_conv9_body, m=m, wp8=wp8, cout=cout, th=th,
                             wd=wd, gram_scale=None)
    out = pl.pallas_call(
        body,
        out_shape=jax.ShapeDtypeStruct((n, nr, m, cout), _DT),
        grid=(n, nr),
        in_specs=[
            pl.BlockSpec((1, 1, l, cin), lambda bi, r: (bi, r, 0, 0)),
            pl.BlockSpec((9 * cin, cout), lambda bi, r: (0, 0)),
            pl.BlockSpec((1, cout), lambda bi, r: (0, 0)),
        ],
        out_specs=pl.BlockSpec((1, 1, m, cout), lambda bi, r: (bi, r, 0, 0)),
        compiler_params=pltpu.CompilerParams(
            dimension_semantics=("parallel", "parallel"),
            vmem_limit_bytes=_VMEM_LIMIT),
    )(x_flat, wf, bias)

    if pool:
        xq = out.reshape(n, h // 2, 2, wp8 // 2, 2 * cout)
        pooled = _maxpool2(xq)
        return pooled[:, :, :wd // 2, :]
    return out.reshape(n, h, wp8, cout)[:, :, :wd, :]


def kernel(x_nchw, w0, b0, w1, b1, w2, b2, w3, b3, w4, b4, w5, b5, w6, b6,
           w7, b7, w8, b8):
    x = jnp.transpose(x_nchw, (0, 2, 3, 1)).astype(_DT)
    cast = lambda w: w.astype(_DT)

    x = _conv1_1(x, cast(w0), b0)
    x = _conv_general(x, cast(w1), b1, pool=True)
    x = _conv_general(x, cast(w2), b2)
    x = _conv_general(x, cast(w3), b3, pool=True)
    x = _conv_general(x, cast(w4), b4)
    x = _conv_general(x, cast(w5), b5)
    x = _conv_general(x, cast(w6), b6)
    x = _conv_general(x, cast(w7), b7, pool=True)
    g = _conv_general(x, cast(w8), b8, gram=True)
    return [g]
